# baseline (speedup 1.0000x reference)
import functools

import jax
import jax.numpy as jnp
from jax import lax
from jax.experimental import pallas as pl
from jax.experimental.pallas import tpu as pltpu
from jax.experimental.pallas import tpu_sc as plsc

N = 10000
HID = 200
HP = 256
HH = 128
E = 320000
STEPS = 6
NC = 2
NS = 16
EPT = E // NS
CH = 125
SEG = 8
NCHS = EPT // (CH * SEG)
NPAIR = NCHS // 2
ACC = 10240
RPT = ACC // NS
WB = 80


_sc_mesh = plsc.VectorSubcoreMesh(core_axis_name="c", subcore_axis_name="s")


@functools.partial(
    pl.kernel,
    out_type=jax.ShapeDtypeStruct((NC, ACC, HH), jnp.float32),
    mesh=_sc_mesh,
    scratch_types=[
        pltpu.VMEM((NCHS, 1, CH), jnp.int32),
        pltpu.VMEM((NCHS, 1, CH), jnp.int32),
        pltpu.VMEM((CH, HH), jnp.float32),
        pltpu.VMEM((CH, HH), jnp.float32),
        pltpu.VMEM_SHARED((ACC, HH), jnp.float32),
        pltpu.SemaphoreType.DMA,
        pltpu.SemaphoreType.DMA,
    ],
)
def _sc_scatter(hsf_hbm, srcs_hbm, dsts_hbm, zrows_hbm, out_hbm,
                src_v, dst_v, rows0, rows1, s_sh, sem0, sem1):
    cid = lax.axis_index("c")
    sid = lax.axis_index("s")

    pltpu.sync_copy(zrows_hbm, rows0.at[pl.ds(0, WB)])
    for t in range(RPT // WB):
        pltpu.sync_copy(rows0.at[pl.ds(0, WB)],
                        s_sh.at[pl.ds(sid * RPT + t * WB, WB)])

    plsc.subcore_barrier()

    for seg in range(SEG):
        pltpu.sync_copy(srcs_hbm.at[cid, sid, seg], src_v)
        pltpu.sync_copy(dsts_hbm.at[sid, seg], dst_v)

        pltpu.async_copy(hsf_hbm.at[src_v.at[0, 0]], rows0, sem0)
        pltpu.async_copy(hsf_hbm.at[src_v.at[1, 0]], rows1, sem1)

        def pair(i, carry):
            j0 = 2 * i
            pltpu.make_async_copy(hsf_hbm.at[src_v.at[j0, 0]], rows0,
                                  sem0).wait()
            pltpu.sync_copy(rows0, s_sh.at[dst_v.at[j0, 0]], add=True)

            @pl.when(i < NPAIR - 1)
            def _():
                pltpu.async_copy(hsf_hbm.at[src_v.at[j0 + 2, 0]], rows0, sem0)

            pltpu.make_async_copy(hsf_hbm.at[src_v.at[j0 + 1, 0]], rows1,
                                  sem1).wait()
            pltpu.sync_copy(rows1, s_sh.at[dst_v.at[j0 + 1, 0]], add=True)

            @pl.when(i < NPAIR - 1)
            def _():
                pltpu.async_copy(hsf_hbm.at[src_v.at[j0 + 3, 0]], rows1, sem1)

            return carry

        lax.fori_loop(0, NPAIR, pair, 0)

    plsc.subcore_barrier()

    for t in range(RPT // WB):
        pltpu.sync_copy(s_sh.at[pl.ds(sid * RPT + t * WB, WB)],
                        rows0.at[pl.ds(0, WB)])
        pltpu.sync_copy(rows0.at[pl.ds(0, WB)],
                        out_hbm.at[cid, pl.ds(sid * RPT + t * WB, WB)])


RB = 1000
NRB = N // RB


H3 = 3 * HP


def _gru_pre_body(h_ref, u_ref, c_ref, out_ref):
    out_ref[...] = jnp.dot(h_ref[...], u_ref[...],
                           preferred_element_type=jnp.float32) + c_ref[...]


def _gru_post_body(h_ref, s_ref, w_ref, wi_ref, bi_ref, hg_ref, out_ref):
    f32 = jnp.float32
    h = h_ref[...]
    agg = jnp.dot(s_ref[...], w_ref[...], preferred_element_type=f32)
    ig = jnp.dot(agg, wi_ref[...], preferred_element_type=f32) + bi_ref[...]
    hg = hg_ref[...]
    r = jax.nn.sigmoid(ig[:, :HP] + hg[:, :HP])
    z = jax.nn.sigmoid(ig[:, HP:2 * HP] + hg[:, HP:2 * HP])
    nn_ = jnp.tanh(ig[:, 2 * HP:] + r * hg[:, 2 * HP:])
    out_ref[...] = (1.0 - z) * nn_ + z * h


_row_spec = pl.BlockSpec((RB, HP), lambda i: (i, 0))
_row3_spec = pl.BlockSpec((RB, H3), lambda i: (i, 0))
_wspec = pl.BlockSpec((HP, HP), lambda i: (0, 0))
_w3spec = pl.BlockSpec((HP, H3), lambda i: (0, 0))
_b3spec = pl.BlockSpec((1, H3), lambda i: (0, 0))

_gru_pre = pl.pallas_call(
    _gru_pre_body,
    grid=(NRB,),
    in_specs=[_row_spec, _w3spec, _b3spec],
    out_specs=_row3_spec,
    out_shape=jax.ShapeDtypeStruct((N, H3), jnp.float32),
)

_gru_post = pl.pallas_call(
    _gru_post_body,
    grid=(NRB,),
    in_specs=[_row_spec, _row_spec, _wspec, _w3spec, _b3spec, _row3_spec],
    out_specs=_row_spec,
    out_shape=jax.ShapeDtypeStruct((N, HP), jnp.float32),
)


def _pool_body(h_ref, out_ref):
    @pl.when(pl.program_id(0) == 0)
    def _init():
        out_ref[...] = jnp.zeros_like(out_ref)

    out_ref[...] += jnp.sum(jax.nn.relu(h_ref[...]), axis=0, keepdims=True)


_pool = pl.pallas_call(
    _pool_body,
    grid=(NRB,),
    in_specs=[_row_spec],
    out_specs=pl.BlockSpec((1, HP), lambda i: (0, 0)),
    out_shape=jax.ShapeDtypeStruct((1, HP), jnp.float32),
)


def _mlp_body(p_ref, w1_ref, b1_ref, w2_ref, b2_ref, w3_ref, b3_ref,
              wc_ref, bc_ref, out_ref):
    f32 = jnp.float32
    a = jax.nn.relu(jnp.dot(p_ref[...], w1_ref[...], preferred_element_type=f32) + b1_ref[...])
    a = jax.nn.relu(jnp.dot(a, w2_ref[...], preferred_element_type=f32) + b2_ref[...])
    a = jax.nn.relu(jnp.dot(a, w3_ref[...], preferred_element_type=f32) + b3_ref[...])
    lg = jnp.dot(a, wc_ref[...], preferred_element_type=f32) + bc_ref[...]
    m = jnp.max(lg, axis=-1, keepdims=True)
    e = jnp.exp(lg - m)
    out_ref[...] = e / jnp.sum(e, axis=-1, keepdims=True)


_mlp = pl.pallas_call(
    _mlp_body,
    out_shape=jax.ShapeDtypeStruct((1, 2), jnp.float32),
)


def _pad_w(w):
    return jnp.pad(w, ((0, HP - HID), (0, HP - HID)))


def kernel(x, edge_index, ggnn_w, w_ih, w_hh, b_ih, b_hh,
           ef1_w, ef1_b, ef2_w, ef2_b, ef3_w, ef3_b, cls_w, cls_b):
    h = jnp.pad(x, ((0, 0), (0, HP - x.shape[1])))

    src = edge_index[0].astype(jnp.int32)
    dst = edge_index[1].astype(jnp.int32)
    src_r = src.reshape(NS, SEG, NCHS, 1, CH)
    srcs = jnp.stack([src_r, src_r + N])
    dsts = dst.reshape(NS, SEG, NCHS, 1, CH)
    zrows = jnp.zeros((WB, HH), dtype=jnp.float32)

    wih_t = w_ih.T
    whh_t = w_hh.T
    wi = jnp.concatenate(
        [_pad_w(wih_t[:, k * HID:(k + 1) * HID]) for k in range(3)], axis=1)
    uh = jnp.concatenate(
        [_pad_w(whh_t[:, k * HID:(k + 1) * HID]) for k in range(3)], axis=1)

    def _pad_b(b):
        return jnp.pad(b, (0, HP - HID)).reshape(1, HP)

    bi = jnp.concatenate(
        [_pad_b(b_ih[k * HID:(k + 1) * HID]) for k in range(3)], axis=1)
    ch_ = jnp.concatenate(
        [_pad_b(b_hh[k * HID:(k + 1) * HID]) for k in range(3)], axis=1)

    for i in range(STEPS):
        hsf = jnp.concatenate([h[:, :HH], h[:, HH:]], axis=0)
        q = _sc_scatter(hsf, srcs, dsts, zrows)
        hg = _gru_pre(h, uh, ch_)
        s = jnp.concatenate([q[0, :N], q[1, :N]], axis=1)
        h = _gru_post(h, s, _pad_w(ggnn_w[i]), wi, bi, hg)

    pooled = _pool(h)
    w1 = jnp.pad(ef1_w.T, ((0, HP - HID), (0, 0)))
    y_a = _mlp(pooled, w1, ef1_b.reshape(1, -1), ef2_w.T,
               ef2_b.reshape(1, -1), ef3_w.T, ef3_b.reshape(1, -1),
               cls_w.T, cls_b.reshape(1, -1))
    return (y_a, x)


# speedup vs baseline: 1.0627x; 1.0627x over previous
import functools

import jax
import jax.numpy as jnp
from jax import lax
from jax.experimental import pallas as pl
from jax.experimental.pallas import tpu as pltpu
from jax.experimental.pallas import tpu_sc as plsc

N = 10000
HID = 200
HP = 256
HH = 128
E = 320000
STEPS = 6
NC = 2
NS = 16
EPT = E // NS
CH = 125
SEG = 8
NCHS = EPT // (CH * SEG)
NPAIR = NCHS // 2
ACC = 10240
RPT = ACC // NS
WB = 80


_sc_mesh = plsc.VectorSubcoreMesh(core_axis_name="c", subcore_axis_name="s")


@functools.partial(
    pl.kernel,
    out_type=jax.ShapeDtypeStruct((ACC, HP), jnp.float32),
    mesh=_sc_mesh,
    scratch_types=[
        pltpu.VMEM((NCHS, 1, CH), jnp.int32),
        pltpu.VMEM((NCHS, 1, CH), jnp.int32),
        pltpu.VMEM((CH, HH), jnp.float32),
        pltpu.VMEM((CH, HH), jnp.float32),
        pltpu.VMEM_SHARED((ACC, HH), jnp.float32),
        pltpu.SemaphoreType.DMA,
        pltpu.SemaphoreType.DMA,
    ],
)
def _sc_scatter(hsf_hbm, srcs_hbm, dsts_hbm, zrows_hbm, out_hbm,
                src_v, dst_v, rows0, rows1, s_sh, sem0, sem1):
    cid = lax.axis_index("c")
    sid = lax.axis_index("s")

    pltpu.sync_copy(zrows_hbm, rows0.at[pl.ds(0, WB)])
    for t in range(RPT // WB):
        pltpu.sync_copy(rows0.at[pl.ds(0, WB)],
                        s_sh.at[pl.ds(sid * RPT + t * WB, WB)])

    plsc.subcore_barrier()

    for seg in range(SEG):
        pltpu.sync_copy(srcs_hbm.at[cid, sid, seg], src_v)
        pltpu.sync_copy(dsts_hbm.at[sid, seg], dst_v)

        pltpu.async_copy(hsf_hbm.at[src_v.at[0, 0]], rows0, sem0)
        pltpu.async_copy(hsf_hbm.at[src_v.at[1, 0]], rows1, sem1)

        def pair(i, carry):
            j0 = 2 * i
            pltpu.make_async_copy(hsf_hbm.at[src_v.at[j0, 0]], rows0,
                                  sem0).wait()
            pltpu.sync_copy(rows0, s_sh.at[dst_v.at[j0, 0]], add=True)

            @pl.when(i < NPAIR - 1)
            def _():
                pltpu.async_copy(hsf_hbm.at[src_v.at[j0 + 2, 0]], rows0, sem0)

            pltpu.make_async_copy(hsf_hbm.at[src_v.at[j0 + 1, 0]], rows1,
                                  sem1).wait()
            pltpu.sync_copy(rows1, s_sh.at[dst_v.at[j0 + 1, 0]], add=True)

            @pl.when(i < NPAIR - 1)
            def _():
                pltpu.async_copy(hsf_hbm.at[src_v.at[j0 + 3, 0]], rows1, sem1)

            return carry

        lax.fori_loop(0, NPAIR, pair, 0)

    plsc.subcore_barrier()

    for t in range(RPT // WB):
        pltpu.sync_copy(s_sh.at[pl.ds(sid * RPT + t * WB, WB)],
                        rows0.at[pl.ds(0, WB)])
        pltpu.sync_copy(rows0.at[pl.ds(0, WB)],
                        out_hbm.at[pl.ds(sid * RPT + t * WB, WB),
                                   pl.ds(cid * HH, HH)])


RB = 1000
NRB = N // RB


G3 = 3 * HH


def _gru_body(hp_ref, h0_ref, h1_ref, s_ref, w_ref,
              wi_ref, bi_ref, uh_ref, ch_ref, out_ref):
    f32 = jnp.float32
    h = jnp.concatenate([h0_ref[...], h1_ref[...]], axis=1)
    agg = jnp.dot(s_ref[...], w_ref[...], preferred_element_type=f32)
    ig = jnp.dot(agg, wi_ref[0], preferred_element_type=f32) + bi_ref[0]
    hg = jnp.dot(h, uh_ref[0], preferred_element_type=f32) + ch_ref[0]
    r = jax.nn.sigmoid(ig[:, :HH] + hg[:, :HH])
    z = jax.nn.sigmoid(ig[:, HH:2 * HH] + hg[:, HH:2 * HH])
    nn_ = jnp.tanh(ig[:, 2 * HH:] + r * hg[:, 2 * HH:])
    out_ref[...] = (1.0 - z) * nn_ + z * hp_ref[...]


_hp_spec = pl.BlockSpec((RB, HH), lambda p, i: (p * NRB + i, 0))
_h0_spec = pl.BlockSpec((RB, HH), lambda p, i: (i, 0))
_h1_spec = pl.BlockSpec((RB, HH), lambda p, i: (NRB + i, 0))
_s_spec = pl.BlockSpec((RB, HP), lambda p, i: (i, 0))
_w_spec = pl.BlockSpec((HP, HP), lambda p, i: (0, 0))
_wi_spec = pl.BlockSpec((1, HP, G3), lambda p, i: (p, 0, 0))
_bi_spec = pl.BlockSpec((1, 1, G3), lambda p, i: (p, 0, 0))

_gru_step = pl.pallas_call(
    _gru_body,
    grid=(2, NRB),
    in_specs=[_hp_spec, _h0_spec, _h1_spec, _s_spec, _w_spec,
              _wi_spec, _bi_spec, _wi_spec, _bi_spec],
    out_specs=_hp_spec,
    out_shape=jax.ShapeDtypeStruct((2 * N, HH), jnp.float32),
)


def _pool_body(h_ref, out_ref):
    @pl.when(pl.program_id(1) == 0)
    def _init():
        out_ref[...] = jnp.zeros_like(out_ref)

    out_ref[...] += jnp.sum(jax.nn.relu(h_ref[...]), axis=0, keepdims=True)


_pool = pl.pallas_call(
    _pool_body,
    grid=(2, NRB),
    in_specs=[_hp_spec],
    out_specs=pl.BlockSpec((8, HH), lambda p, i: (p, 0)),
    out_shape=jax.ShapeDtypeStruct((16, HH), jnp.float32),
)


def _mlp_body(p_ref, w1_ref, b1_ref, w2_ref, b2_ref, w3_ref, b3_ref,
              wc_ref, bc_ref, out_ref):
    f32 = jnp.float32
    a = jax.nn.relu(jnp.dot(p_ref[...], w1_ref[...], preferred_element_type=f32) + b1_ref[...])
    a = jax.nn.relu(jnp.dot(a, w2_ref[...], preferred_element_type=f32) + b2_ref[...])
    a = jax.nn.relu(jnp.dot(a, w3_ref[...], preferred_element_type=f32) + b3_ref[...])
    lg = jnp.dot(a, wc_ref[...], preferred_element_type=f32) + bc_ref[...]
    m = jnp.max(lg, axis=-1, keepdims=True)
    e = jnp.exp(lg - m)
    out_ref[...] = e / jnp.sum(e, axis=-1, keepdims=True)


_mlp = pl.pallas_call(
    _mlp_body,
    out_shape=jax.ShapeDtypeStruct((1, 2), jnp.float32),
)


def _pad_w(w):
    return jnp.pad(w, ((0, HP - HID), (0, HP - HID)))


def kernel(x, edge_index, ggnn_w, w_ih, w_hh, b_ih, b_hh,
           ef1_w, ef1_b, ef2_w, ef2_b, ef3_w, ef3_b, cls_w, cls_b):
    xp = jnp.pad(x, ((0, 0), (0, HP - x.shape[1])))
    hs = jnp.concatenate([xp[:, :HH], xp[:, HH:]], axis=0)

    src = edge_index[0].astype(jnp.int32)
    dst = edge_index[1].astype(jnp.int32)
    src_r = src.reshape(NS, SEG, NCHS, 1, CH)
    srcs = jnp.stack([src_r, src_r + N])
    dsts = dst.reshape(NS, SEG, NCHS, 1, CH)
    zrows = jnp.zeros((WB, HH), dtype=jnp.float32)

    wih_t = w_ih.T
    whh_t = w_hh.T
    wg = [_pad_w(wih_t[:, k * HID:(k + 1) * HID]) for k in range(3)]
    ug = [_pad_w(whh_t[:, k * HID:(k + 1) * HID]) for k in range(3)]

    def _pad_b(b):
        return jnp.pad(b, (0, HP - HID)).reshape(1, HP)

    bg = [_pad_b(b_ih[k * HID:(k + 1) * HID]) for k in range(3)]
    cg = [_pad_b(b_hh[k * HID:(k + 1) * HID]) for k in range(3)]

    def _plane_pack(mats):
        return jnp.stack([
            jnp.concatenate([m[:, p * HH:(p + 1) * HH] for m in mats], axis=1)
            for p in range(2)])

    wi3, uh3 = _plane_pack(wg), _plane_pack(ug)
    bi3, ch3 = _plane_pack(bg), _plane_pack(cg)

    for i in range(STEPS):
        q = _sc_scatter(hs, srcs, dsts, zrows)
        hs = _gru_step(hs, hs, hs, q, _pad_w(ggnn_w[i]), wi3, bi3, uh3, ch3)

    pq = _pool(hs)
    pooled = jnp.concatenate([pq[0:1], pq[8:9]], axis=1)
    w1 = jnp.pad(ef1_w.T, ((0, HP - HID), (0, 0)))
    y_a = _mlp(pooled, w1, ef1_b.reshape(1, -1), ef2_w.T,
               ef2_b.reshape(1, -1), ef3_w.T, ef3_b.reshape(1, -1),
               cls_w.T, cls_b.reshape(1, -1))
    return (y_a, x)
